# Initial kernel scaffold; baseline (speedup 1.0000x reference)
import jax
import jax.numpy as jnp
from jax.experimental import pallas as pl


def kernel(features, coords, edge_index, W1, b1, W2, b2, W3, b3, W4, b4, W5, b5):
    raise NotImplementedError("write your pallas kernel here")


# speedup vs baseline: 4.8933x; 4.8933x over previous
import functools

import jax
import jax.numpy as jnp
from jax import lax
from jax.experimental import pallas as pl
from jax.experimental.pallas import tpu as pltpu
from jax.experimental.pallas import tpu_sc as plsc

N = 10000
E = 320000
D = 128
H = 64
WID = 128

NC = 2
NS = 16
NW = NC * NS
EPW = E // NW
CH = 80
NCHUNK = EPW // CH

BE = 4000


def _silu(x):
    return x * jax.nn.sigmoid(x)


def _prep_body(f_ref, c64_ref, w1a_ref, w1b_ref, rr_ref, rc_ref):
    f = f_ref[...]
    rr_ref[:, :H] = jnp.dot(f, w1a_ref[...], preferred_element_type=jnp.float32)
    rr_ref[:, H:] = c64_ref[...]
    rc_ref[:, :H] = jnp.dot(f, w1b_ref[...], preferred_element_type=jnp.float32)
    rc_ref[:, H:] = c64_ref[...]


def _prep(features, coords64, w1a, w1b):
    return pl.pallas_call(
        _prep_body,
        out_shape=[
            jax.ShapeDtypeStruct((N, WID), jnp.float32),
            jax.ShapeDtypeStruct((N, WID), jnp.float32),
        ],
    )(features, coords64, w1a, w1b)


def _gather_body(rr_hbm, rc_hbm, row_hbm, col_hbm, zr_hbm, zc_hbm,
                 idxr_v, idxc_v, bufr_v, bufc_v, semr, semc):
    wid = lax.axis_index("s") * NC + lax.axis_index("c")
    base = wid * EPW

    def body(i, carry):
        off = base + i * CH
        pltpu.sync_copy(row_hbm.at[pl.ds(off, CH)], idxr_v)
        pltpu.sync_copy(col_hbm.at[pl.ds(off, CH)], idxc_v)
        cr = pltpu.async_copy(rr_hbm.at[idxr_v], bufr_v, semr)
        cc = pltpu.async_copy(rc_hbm.at[idxc_v], bufc_v, semc)
        cr.wait()
        cc.wait()
        pltpu.sync_copy(bufr_v, zr_hbm.at[pl.ds(off, CH)])
        pltpu.sync_copy(bufc_v, zc_hbm.at[pl.ds(off, CH)])
        return carry

    lax.fori_loop(0, NCHUNK, body, 0)


def _gather(rr, rc, row, col):
    mesh = plsc.VectorSubcoreMesh(
        core_axis_name="c", subcore_axis_name="s", num_cores=NC, num_subcores=NS)
    kern = functools.partial(
        pl.kernel,
        out_type=[
            jax.ShapeDtypeStruct((E, WID), jnp.float32),
            jax.ShapeDtypeStruct((E, WID), jnp.float32),
        ],
        mesh=mesh,
        scratch_types=[
            pltpu.VMEM((CH,), jnp.int32),
            pltpu.VMEM((CH,), jnp.int32),
            pltpu.VMEM((CH, WID), jnp.float32),
            pltpu.VMEM((CH, WID), jnp.float32),
            pltpu.SemaphoreType.DMA,
            pltpu.SemaphoreType.DMA,
        ],
    )(_gather_body)
    return kern(rr, rc, row, col)


def _edge_body(zr_ref, zc_ref, w1c_ref, b1_ref, w2_ref, b2_ref,
               w3_ref, b3_ref, w4_ref, b4_ref, out_ref):
    zr = zr_ref[:, :H]
    zc = zc_ref[:, :H]
    diff16 = zr_ref[:, H:] - zc_ref[:, H:]
    dist = jnp.sum(diff16 * diff16, axis=1, keepdims=True)
    m = _silu(zr + zc + dist * w1c_ref[...] + b1_ref[...])
    msg = _silu(jnp.dot(m, w2_ref[...], preferred_element_type=jnp.float32)
                + b2_ref[...])
    c3 = _silu(jnp.dot(msg, w3_ref[...], preferred_element_type=jnp.float32)
               + b3_ref[...])
    cw = jnp.sum(c3 * w4_ref[...], axis=1, keepdims=True) + b4_ref[...]
    out_ref[:, :H] = msg
    out_ref[:, H:] = cw * diff16


def _edge_mlp(zr, zc, w1c, b1, w2, b2, w3, b3, w4r, b4):
    nblk = E // BE
    blk = lambda i: (i, 0)
    fixed = lambda i: (0, 0)
    return pl.pallas_call(
        _edge_body,
        grid=(nblk,),
        in_specs=[
            pl.BlockSpec((BE, WID), blk),
            pl.BlockSpec((BE, WID), blk),
            pl.BlockSpec((1, H), fixed),
            pl.BlockSpec((1, H), fixed),
            pl.BlockSpec((H, H), fixed),
            pl.BlockSpec((1, H), fixed),
            pl.BlockSpec((H, H), fixed),
            pl.BlockSpec((1, H), fixed),
            pl.BlockSpec((1, H), fixed),
            pl.BlockSpec((1, 1), fixed),
        ],
        out_specs=pl.BlockSpec((BE, WID), blk),
        out_shape=jax.ShapeDtypeStruct((E, WID), jnp.float32),
    )(zr, zc, w1c, b1, w2, b2, w3, b3, w4r, b4)


def _scatter_body(u_hbm, row_hbm, zeros_hbm, out_hbm, idx_v, buf_v, acc_sh):
    cid = lax.axis_index("c")
    sid = lax.axis_index("s")
    wid = sid * NC + cid
    base = wid * EPW

    @pl.when(sid == 0)
    def _():
        pltpu.sync_copy(zeros_hbm, acc_sh)

    plsc.subcore_barrier()

    def body(i, carry):
        off = base + i * CH
        pltpu.sync_copy(row_hbm.at[pl.ds(off, CH)], idx_v)
        pltpu.sync_copy(u_hbm.at[pl.ds(off, CH)], buf_v)
        pltpu.sync_copy(buf_v, acc_sh.at[idx_v], add=True)
        return carry

    lax.fori_loop(0, NCHUNK, body, 0)
    plsc.subcore_barrier()

    @pl.when(sid == 0)
    def _():
        pltpu.sync_copy(acc_sh, out_hbm.at[cid])


def _scatter(u, row, zeros):
    mesh = plsc.VectorSubcoreMesh(
        core_axis_name="c", subcore_axis_name="s", num_cores=NC, num_subcores=NS)
    kern = functools.partial(
        pl.kernel,
        out_type=jax.ShapeDtypeStruct((NC, N, WID), jnp.float32),
        mesh=mesh,
        scratch_types=[
            pltpu.VMEM((CH,), jnp.int32),
            pltpu.VMEM((CH, WID), jnp.float32),
            pltpu.VMEM_SHARED((N, WID), jnp.float32),
        ],
    )(_scatter_body)
    return kern(u, row, zeros)


def _final_body(f_ref, c64_ref, scat_ref, w5a_ref, w5b_ref, b5_ref,
                nf_ref, nc64_ref):
    f = f_ref[...]
    agg = scat_ref[0, :, :H] + scat_ref[1, :, :H]
    pre = (jnp.dot(f, w5a_ref[...], preferred_element_type=jnp.float32)
           + jnp.dot(agg, w5b_ref[...], preferred_element_type=jnp.float32)
           + b5_ref[...])
    nf_ref[...] = _silu(pre)
    nc64_ref[...] = c64_ref[...] + scat_ref[0, :, H:] + scat_ref[1, :, H:]


def _final(features, coords64, scat, w5a, w5b, b5):
    return pl.pallas_call(
        _final_body,
        out_shape=[
            jax.ShapeDtypeStruct((N, D), jnp.float32),
            jax.ShapeDtypeStruct((N, H), jnp.float32),
        ],
    )(features, coords64, scat, w5a, w5b, b5)


def kernel(features, coords, edge_index, W1, b1, W2, b2, W3, b3, W4, b4, W5, b5):
    row = edge_index[0].astype(jnp.int32)
    col = edge_index[1].astype(jnp.int32)
    coords64 = jnp.pad(coords, ((0, 0), (0, 61)))

    w1a = W1[:D]
    w1b = W1[D:2 * D]
    w1c = W1[2 * D].reshape(1, H)

    rr, rc = _prep(features, coords64, w1a, w1b)
    zr, zc = _gather(rr, rc, row, col)
    payload = _edge_mlp(
        zr, zc, w1c, b1.reshape(1, H), W2, b2.reshape(1, H),
        W3, b3.reshape(1, H), W4.reshape(1, H), b4.reshape(1, 1))
    scat = _scatter(payload, row, jnp.zeros((N, WID), jnp.float32))
    new_features, nc64 = _final(features, coords64, scat, W5[:D], W5[D:], b5.reshape(1, D))
    return (new_features, nc64[:, :3])
